# Initial kernel scaffold; baseline (speedup 1.0000x reference)
import jax
import jax.numpy as jnp
from jax.experimental import pallas as pl


def kernel(lookup, embedding_table):
    raise NotImplementedError("write your pallas kernel here")


# speedup vs baseline: 1.0225x; 1.0225x over previous
import functools

import jax
import jax.numpy as jnp
from jax import lax
from jax.experimental import pallas as pl
from jax.experimental.pallas import tpu as pltpu
from jax.experimental.pallas import tpu_sc as plsc


def _gather_kernel(n_total, n_per_w, chunk, d):
    n_chunks = n_per_w // chunk
    mesh = plsc.VectorSubcoreMesh(core_axis_name="c", subcore_axis_name="s")

    @functools.partial(
        pl.kernel,
        mesh=mesh,
        out_type=jax.ShapeDtypeStruct((n_total, d), jnp.float32),
        scratch_types=[
            pltpu.VMEM((n_per_w,), jnp.int32),
            pltpu.VMEM((chunk, d), jnp.float32),
            pltpu.SemaphoreType.DMA,
        ],
        compiler_params=pltpu.CompilerParams(use_tc_tiling_on_sc=False),
    )
    def k(idx_hbm, tab_hbm, out_hbm, idx_v, rows_v, sem):
        wid = lax.axis_index("s") * 2 + lax.axis_index("c")
        base = wid * n_per_w
        pltpu.sync_copy(idx_hbm.at[pl.ds(base, n_per_w)], idx_v)

        def body(j, carry):
            off = j * chunk
            pltpu.async_copy(
                tab_hbm.at[idx_v.at[pl.ds(off, chunk)]], rows_v, sem
            ).wait()
            pltpu.sync_copy(rows_v, out_hbm.at[pl.ds(base + off, chunk)])
            return carry

        lax.fori_loop(0, n_chunks, body, 0)

    return k


def kernel(lookup, embedding_table):
    b, h = lookup.shape
    v, d = embedding_table.shape
    idx = lookup.reshape(-1).astype(jnp.int32)
    n_total = b * h
    n_workers = 32
    n_per_w = n_total // n_workers
    out = _gather_kernel(n_total, n_per_w, 128, d)(idx, embedding_table)
    return out.reshape(b, h, d)


# speedup vs baseline: 1.1132x; 1.0888x over previous
import functools

import jax
import jax.numpy as jnp
from jax import lax
from jax.experimental import pallas as pl
from jax.experimental.pallas import tpu as pltpu
from jax.experimental.pallas import tpu_sc as plsc

_NBUF = 4


def _gather_kernel(n_total, n_per_w, chunk, d):
    n_chunks = n_per_w // chunk
    n_outer = n_chunks // _NBUF
    mesh = plsc.VectorSubcoreMesh(core_axis_name="c", subcore_axis_name="s")

    @functools.partial(
        pl.kernel,
        mesh=mesh,
        out_type=jax.ShapeDtypeStruct((n_total, d), jnp.float32),
        scratch_types=(
            [pltpu.VMEM((n_per_w,), jnp.int32)]
            + [pltpu.VMEM((chunk, d), jnp.float32) for _ in range(_NBUF)]
            + [pltpu.SemaphoreType.DMA for _ in range(2 * _NBUF)]
        ),
        compiler_params=pltpu.CompilerParams(use_tc_tiling_on_sc=False),
    )
    def k(idx_hbm, tab_hbm, out_hbm, idx_v, *bufs_and_sems):
        bufs = bufs_and_sems[:_NBUF]
        sem_g = bufs_and_sems[_NBUF : 2 * _NBUF]
        sem_s = bufs_and_sems[2 * _NBUF :]
        wid = lax.axis_index("s") * 2 + lax.axis_index("c")
        base = wid * n_per_w
        pltpu.sync_copy(idx_hbm.at[pl.ds(base, n_per_w)], idx_v)

        def start_gather(j, b):
            pltpu.async_copy(
                tab_hbm.at[idx_v.at[pl.ds(j * chunk, chunk)]], bufs[b], sem_g[b]
            )

        def wait_gather(b):
            pltpu.make_async_copy(
                out_hbm.at[pl.ds(0, chunk)], bufs[b], sem_g[b]
            ).wait()

        def start_store(j, b):
            pltpu.async_copy(
                bufs[b], out_hbm.at[pl.ds(base + j * chunk, chunk)], sem_s[b]
            )

        def wait_store(b):
            pltpu.make_async_copy(
                bufs[b], out_hbm.at[pl.ds(0, chunk)], sem_s[b]
            ).wait()

        for b in range(_NBUF - 1):
            start_gather(b, b)

        def outer(g, carry):
            for b in range(_NBUF):
                j = g * _NBUF + b
                pb = (b - 1) % _NBUF
                wait_gather(b)
                start_store(j, b)

                @pl.when(j > 0)
                def _():
                    wait_store(pb)

                @pl.when(j + _NBUF - 1 < n_chunks)
                def _():
                    start_gather(j + _NBUF - 1, pb)

            return carry

        lax.fori_loop(0, n_outer, outer, 0)
        wait_store((n_chunks - 1) % _NBUF)

    return k


def kernel(lookup, embedding_table):
    b, h = lookup.shape
    v, d = embedding_table.shape
    idx = lookup.reshape(-1).astype(jnp.int32)
    n_total = b * h
    n_workers = 32
    n_per_w = n_total // n_workers
    out = _gather_kernel(n_total, n_per_w, 640, d)(idx, embedding_table)
    return out.reshape(b, h, d)


# speedup vs baseline: 1.7931x; 1.6107x over previous
import functools

import jax
import jax.numpy as jnp
from jax import lax
from jax.experimental import pallas as pl
from jax.experimental.pallas import tpu as pltpu
from jax.experimental.pallas import tpu_sc as plsc

_NBUF = 4
_NW = 32
_CB = 8
_L = 16


def _gather_kernel(bsz, h, d):
    rows_per_w = bsz // _NW
    n_chunks = rows_per_w // _CB
    n_outer = n_chunks // _NBUF
    chunk = _CB * h
    n_spans = chunk // _L
    mesh = plsc.VectorSubcoreMesh(core_axis_name="c", subcore_axis_name="s")

    @functools.partial(
        pl.kernel,
        mesh=mesh,
        out_type=jax.ShapeDtypeStruct((bsz, h, d), jnp.float32),
        scratch_types=(
            [
                pltpu.VMEM((rows_per_w, h), jnp.int32),
                pltpu.VMEM((n_chunks, chunk), jnp.int32),
            ]
            + [pltpu.VMEM((chunk, d), jnp.float32) for _ in range(_NBUF)]
            + [pltpu.SemaphoreType.DMA for _ in range(2 * _NBUF)]
        ),
        compiler_params=pltpu.CompilerParams(
            use_tc_tiling_on_sc=False, needs_layout_passes=False
        ),
    )
    def k(lookup_hbm, tab_hbm, out_hbm, idx2_v, idx_v, *bufs_and_sems):
        bufs = bufs_and_sems[:_NBUF]
        sem_g = bufs_and_sems[_NBUF : 2 * _NBUF]
        sem_s = bufs_and_sems[2 * _NBUF :]
        wid = lax.axis_index("s") * 2 + lax.axis_index("c")
        rbase = wid * rows_per_w
        pltpu.sync_copy(lookup_hbm.at[pl.ds(rbase, rows_per_w)], idx2_v)

        lanes = jax.lax.iota(jnp.int32, _L)

        def flatten(g, carry):
            r0 = g * _CB
            for k_ in range(n_spans):
                pos = k_ * _L + lanes
                rk = lax.div(pos, h)
                ck = pos - rk * h
                vals = plsc.load_gather(idx2_v, [r0 + rk, ck])
                idx_v[g, pl.ds(k_ * _L, _L)] = vals
            return carry

        lax.fori_loop(0, n_chunks, flatten, 0)

        def start_gather(j, b):
            pltpu.async_copy(tab_hbm.at[idx_v.at[j]], bufs[b], sem_g[b])

        def wait_gather(b):
            pltpu.make_async_copy(
                tab_hbm.at[pl.ds(0, chunk)], bufs[b], sem_g[b]
            ).wait()

        def start_store(j, b):
            for r in range(_CB):
                pltpu.async_copy(
                    bufs[b].at[pl.ds(r * h, h)],
                    out_hbm.at[rbase + j * _CB + r],
                    sem_s[b],
                )

        def wait_store(b):
            pltpu.make_async_copy(
                bufs[b], tab_hbm.at[pl.ds(0, chunk)], sem_s[b]
            ).wait()

        for b in range(_NBUF - 1):
            start_gather(b, b)

        def outer(g, carry):
            for b in range(_NBUF):
                j = g * _NBUF + b
                pb = (b - 1) % _NBUF
                wait_gather(b)
                start_store(j, b)

                @pl.when(j > 0)
                def _():
                    wait_store(pb)

                @pl.when(j + _NBUF - 1 < n_chunks)
                def _():
                    start_gather(j + _NBUF - 1, pb)

            return carry

        lax.fori_loop(0, n_outer, outer, 0)
        wait_store((n_chunks - 1) % _NBUF)

    return k


def kernel(lookup, embedding_table):
    bsz, h = lookup.shape
    v, d = embedding_table.shape
    return _gather_kernel(bsz, h, d)(lookup, embedding_table)


# speedup vs baseline: 1.7985x; 1.0030x over previous
import functools

import jax
import jax.numpy as jnp
from jax import lax
from jax.experimental import pallas as pl
from jax.experimental.pallas import tpu as pltpu
from jax.experimental.pallas import tpu_sc as plsc

_NBUF = 4
_NW = 32
_CB = 8
_L = 16
_W = 128
_HP = 56


def _gather_kernel(bsz, h, d):
    rows_per_w = bsz // _NW
    n_chunks = rows_per_w // _CB
    n_outer = n_chunks // _NBUF
    chunk = _CB * h
    n_spans = chunk // _L
    mesh = plsc.VectorSubcoreMesh(core_axis_name="c", subcore_axis_name="s")

    @functools.partial(
        pl.kernel,
        mesh=mesh,
        out_type=jax.ShapeDtypeStruct((bsz, h, d), jnp.float32),
        scratch_types=(
            [
                pltpu.VMEM((rows_per_w, _HP), jnp.int32),
                pltpu.VMEM((n_chunks, chunk), jnp.int32),
            ]
            + [pltpu.VMEM((chunk, d), jnp.float32) for _ in range(_NBUF)]
            + [pltpu.SemaphoreType.DMA for _ in range(2 * _NBUF)]
        ),
        compiler_params=pltpu.CompilerParams(
            use_tc_tiling_on_sc=False, needs_layout_passes=False
        ),
    )
    def k(lookup_hbm, tab_hbm, out_hbm, idx2_v, idx_v, *bufs_and_sems):
        bufs = bufs_and_sems[:_NBUF]
        sem_g = bufs_and_sems[_NBUF : 2 * _NBUF]
        sem_s = bufs_and_sems[2 * _NBUF :]
        wid = lax.axis_index("s") * 2 + lax.axis_index("c")
        rbase = wid * rows_per_w
        pltpu.sync_copy(
            lookup_hbm.at[pl.ds(rbase, rows_per_w), pl.ds(0, _HP)], idx2_v
        )

        lanes = jax.lax.iota(jnp.int32, _L)

        def flatten(g, carry):
            r0 = g * _CB
            for k_ in range(n_spans):
                pos = k_ * _L + lanes
                rk = lax.div(pos, h)
                ck = pos - rk * h
                vals = plsc.load_gather(idx2_v, [r0 + rk, ck])
                idx_v[g, pl.ds(k_ * _L, _L)] = vals
            return carry

        lax.fori_loop(0, n_chunks, flatten, 0)

        def start_gather(j, b):
            pltpu.async_copy(tab_hbm.at[idx_v.at[j]], bufs[b], sem_g[b])

        def wait_gather(b):
            pltpu.make_async_copy(
                tab_hbm.at[pl.ds(0, chunk)], bufs[b], sem_g[b]
            ).wait()

        def start_store(j, b):
            for r in range(_CB):
                pltpu.async_copy(
                    bufs[b].at[pl.ds(r * h, h)],
                    out_hbm.at[rbase + j * _CB + r],
                    sem_s[b],
                )

        def wait_store(b):
            pltpu.make_async_copy(
                bufs[b], tab_hbm.at[pl.ds(0, chunk)], sem_s[b]
            ).wait()

        for b in range(_NBUF - 1):
            start_gather(b, b)

        def outer(g, carry):
            for b in range(_NBUF):
                j = g * _NBUF + b
                pb = (b - 1) % _NBUF
                wait_gather(b)
                start_store(j, b)

                @pl.when(j > 0)
                def _():
                    wait_store(pb)

                @pl.when(j + _NBUF - 1 < n_chunks)
                def _():
                    start_gather(j + _NBUF - 1, pb)

            return carry

        lax.fori_loop(0, n_outer, outer, 0)
        wait_store((n_chunks - 1) % _NBUF)

    return k


def kernel(lookup, embedding_table):
    bsz, h = lookup.shape
    v, d = embedding_table.shape
    lookup_pad = jnp.pad(lookup, ((0, 0), (0, _W - h)))
    return _gather_kernel(bsz, h, d)(lookup_pad, embedding_table)


# speedup vs baseline: 2.5251x; 1.4040x over previous
import functools

import jax
import jax.numpy as jnp
from jax import lax
from jax.experimental import pallas as pl
from jax.experimental.pallas import tpu as pltpu
from jax.experimental.pallas import tpu_sc as plsc

_NBUF = 4
_NW = 32
_CB = 8
_L = 16
_W = 128
_HP = 56


def _gather_kernel(bsz, h, d):
    rows_per_w = bsz // _NW
    n_chunks = rows_per_w // _CB
    n_outer = n_chunks // _NBUF
    chunk = _CB * h
    n_spans = chunk // _L
    mesh = plsc.VectorSubcoreMesh(core_axis_name="c", subcore_axis_name="s")

    @functools.partial(
        pl.kernel,
        mesh=mesh,
        out_type=jax.ShapeDtypeStruct((bsz, _HP, _W), jnp.float32),
        scratch_types=(
            [
                pltpu.VMEM((rows_per_w, _HP), jnp.int32),
                pltpu.VMEM((n_chunks, chunk), jnp.int32),
            ]
            + [pltpu.VMEM((chunk, d), jnp.float32) for _ in range(_NBUF)]
            + [pltpu.SemaphoreType.DMA for _ in range(2 * _NBUF)]
        ),
        compiler_params=pltpu.CompilerParams(
            use_tc_tiling_on_sc=False, needs_layout_passes=False
        ),
    )
    def k(lookup_hbm, tab_hbm, out_hbm, idx2_v, idx_v, *bufs_and_sems):
        bufs = bufs_and_sems[:_NBUF]
        sem_g = bufs_and_sems[_NBUF : 2 * _NBUF]
        sem_s = bufs_and_sems[2 * _NBUF :]
        wid = lax.axis_index("s") * 2 + lax.axis_index("c")
        rbase = wid * rows_per_w
        pltpu.sync_copy(
            lookup_hbm.at[pl.ds(rbase, rows_per_w), pl.ds(0, _HP)], idx2_v
        )

        lanes = jax.lax.iota(jnp.int32, _L)

        def flatten(g, carry):
            r0 = g * _CB
            for k_ in range(n_spans):
                pos = k_ * _L + lanes
                rk = lax.div(pos, h)
                ck = pos - rk * h
                vals = plsc.load_gather(idx2_v, [r0 + rk, ck])
                idx_v[g, pl.ds(k_ * _L, _L)] = vals
            return carry

        lax.fori_loop(0, n_chunks, flatten, 0)

        def start_gather(j, b):
            pltpu.async_copy(tab_hbm.at[idx_v.at[j]], bufs[b], sem_g[b])

        def wait_gather(b):
            pltpu.make_async_copy(
                tab_hbm.at[pl.ds(0, chunk)], bufs[b], sem_g[b]
            ).wait()

        def start_store(j, b):
            for r in range(_CB):
                pltpu.async_copy(
                    bufs[b].at[pl.ds(r * h, h)],
                    out_hbm.at[rbase + j * _CB + r, pl.ds(0, h), pl.ds(0, d)],
                    sem_s[b],
                )

        def wait_store(b):
            pltpu.make_async_copy(
                bufs[b], tab_hbm.at[pl.ds(0, chunk)], sem_s[b]
            ).wait()

        for b in range(_NBUF - 1):
            start_gather(b, b)

        def outer(g, carry):
            for b in range(_NBUF):
                j = g * _NBUF + b
                pb = (b - 1) % _NBUF
                wait_gather(b)
                start_store(j, b)

                @pl.when(j > 0)
                def _():
                    wait_store(pb)

                @pl.when(j + _NBUF - 1 < n_chunks)
                def _():
                    start_gather(j + _NBUF - 1, pb)

            return carry

        lax.fori_loop(0, n_outer, outer, 0)
        wait_store((n_chunks - 1) % _NBUF)

    return k


def kernel(lookup, embedding_table):
    bsz, h = lookup.shape
    v, d = embedding_table.shape
    lookup_pad = jnp.pad(lookup, ((0, 0), (0, _W - h)))
    out = _gather_kernel(bsz, h, d)(lookup_pad, embedding_table)
    return out[:, :h, :d]


# speedup vs baseline: 2.5272x; 1.0008x over previous
import functools

import jax
import jax.numpy as jnp
from jax import lax
from jax.experimental import pallas as pl
from jax.experimental.pallas import tpu as pltpu
from jax.experimental.pallas import tpu_sc as plsc

_NBUF = 4
_NW = 32
_CB = 8
_L = 16
_W = 128


def _gather_kernel(bsz, h, d):
    hp = -(-h // 8) * 8
    rows_per_w = bsz // _NW
    n_chunks = rows_per_w // _CB
    n_outer = n_chunks // _NBUF
    chunk = _CB * h
    n_spans = chunk // _L
    mesh = plsc.VectorSubcoreMesh(core_axis_name="c", subcore_axis_name="s")

    @functools.partial(
        pl.kernel,
        mesh=mesh,
        out_type=jax.ShapeDtypeStruct((bsz, hp, _W), jnp.float32),
        scratch_types=(
            [
                pltpu.VMEM((rows_per_w, hp), jnp.int32),
                pltpu.VMEM((n_chunks, chunk), jnp.int32),
            ]
            + [pltpu.VMEM((chunk, d), jnp.float32) for _ in range(_NBUF)]
            + [pltpu.SemaphoreType.DMA for _ in range(2 * _NBUF)]
        ),
        compiler_params=pltpu.CompilerParams(
            use_tc_tiling_on_sc=False, needs_layout_passes=False
        ),
    )
    def k(lookup_hbm, tab_hbm, out_hbm, idx2_v, idx_v, *bufs_and_sems):
        bufs = bufs_and_sems[:_NBUF]
        sem_g = bufs_and_sems[_NBUF : 2 * _NBUF]
        sem_s = bufs_and_sems[2 * _NBUF :]
        wid = lax.axis_index("s") * 2 + lax.axis_index("c")
        rbase = wid * rows_per_w
        pltpu.sync_copy(
            lookup_hbm.at[pl.ds(rbase, rows_per_w), pl.ds(0, hp)], idx2_v
        )

        lanes = jax.lax.iota(jnp.int32, _L)

        def flatten(g, carry):
            r0 = g * _CB
            for k_ in range(n_spans):
                pos = k_ * _L + lanes
                rk = lax.div(pos, h)
                ck = pos - rk * h
                vals = plsc.load_gather(idx2_v, [r0 + rk, ck])
                idx_v[g, pl.ds(k_ * _L, _L)] = vals
            return carry

        lax.fori_loop(0, n_chunks, flatten, 0)

        def start_gather(j, b):
            pltpu.async_copy(tab_hbm.at[idx_v.at[j]], bufs[b], sem_g[b])

        def wait_gather(b):
            pltpu.make_async_copy(
                tab_hbm.at[pl.ds(0, chunk)], bufs[b], sem_g[b]
            ).wait()

        def start_store(j, b):
            for r in range(_CB):
                pltpu.async_copy(
                    bufs[b].at[pl.ds(r * h, h)],
                    out_hbm.at[rbase + j * _CB + r, pl.ds(0, h), pl.ds(0, d)],
                    sem_s[b],
                )

        def wait_store(b):
            pltpu.make_async_copy(
                bufs[b], tab_hbm.at[pl.ds(0, chunk)], sem_s[b]
            ).wait()

        for b in range(_NBUF - 1):
            start_gather(b, b)

        def outer(g, carry):
            for b in range(_NBUF):
                j = g * _NBUF + b
                pb = (b - 1) % _NBUF
                wait_gather(b)
                start_store(j, b)

                @pl.when(j > 0)
                def _():
                    wait_store(pb)

                @pl.when(j + _NBUF - 1 < n_chunks)
                def _():
                    start_gather(j + _NBUF - 1, pb)

            return carry

        lax.fori_loop(0, n_outer, outer, 0)
        wait_store((n_chunks - 1) % _NBUF)

    return k


def kernel(lookup, embedding_table):
    bsz, h = lookup.shape
    v, d = embedding_table.shape
    lookup_pad = jnp.pad(lookup, ((0, 0), (0, _W - h)))
    out = _gather_kernel(bsz, h, d)(lookup_pad, embedding_table)
    return out[:, :h, :d]
